# Initial kernel scaffold; baseline (speedup 1.0000x reference)
import jax
import jax.numpy as jnp
from jax.experimental import pallas as pl


def kernel(node_feats, edge_index, W0p, b0p, W0, b0, bias0, W1p, b1p, W1, b1, bias1, W2p, b2p, W2, b2, bias2):
    raise NotImplementedError("write your pallas kernel here")


# speedup vs baseline: 1.3566x; 1.3566x over previous
import functools

import jax
import jax.numpy as jnp
from jax import lax
from jax.experimental import pallas as pl
from jax.experimental.pallas import tpu as pltpu
from jax.experimental.pallas import tpu_sc as plsc

N = 10000
NPAD = 10240
NW = 32
NPW = NPAD // NW
E = 320000
ECHUNK = 4000
NSTEPS = ECHUNK // 16
GCH = 64


def _make_segmax(F):
  mesh = plsc.VectorSubcoreMesh(core_axis_name="c", subcore_axis_name="s")

  @functools.partial(
      pl.kernel,
      out_type=jax.ShapeDtypeStruct((NPAD, F), jnp.float32),
      mesh=mesh,
      scratch_types=[
          pltpu.VMEM((NPW + 1, F), jnp.float32),
          pltpu.VMEM((ECHUNK,), jnp.int32),
          pltpu.VMEM((ECHUNK,), jnp.int32),
          pltpu.VMEM((GCH + 16,), jnp.int32),
          pltpu.VMEM((GCH + 16,), jnp.int32),
          pltpu.VMEM((GCH, F), jnp.float32),
          pltpu.SemaphoreType.DMA,
      ],
      compiler_params=pltpu.CompilerParams(needs_layout_passes=False),
  )
  def segmax(hp_hbm, src_hbm, dst_hbm, out_hbm,
             agg, srcv, dstv, gidx, gdst, rows, gsem):
    c = lax.axis_index("c")
    s = lax.axis_index("s")
    wid = s * 2 + c
    lo = wid * NPW
    iot = lax.iota(jnp.int32, 16)

    zf = jnp.zeros((16,), jnp.float32)
    zi = jnp.zeros((16,), jnp.int32)

    def zero_row(r, carry):
      for fb in range(F // 16):
        agg[r, pl.ds(fb * 16, 16)] = zf
      return carry

    lax.fori_loop(0, NPW + 1, zero_row, 0)

    for i in range((GCH + 16) // 16):
      gidx[pl.ds(i * 16, 16)] = zi

    def flush(nvalid):
      pltpu.async_copy(hp_hbm.at[gidx.at[pl.ds(0, GCH)]], rows, gsem).wait()

      def group_body(g, carry):
        dv = gdst[pl.ds(g * 16, 16)]
        base = g * 16
        for e in range(16):
          ldx = jnp.max(jnp.where(iot == e, dv, jnp.int32(-1)))
          ldx = jnp.where(base + e < nvalid, ldx, jnp.int32(NPW))
          re = base + e
          for fb in range(F // 16):
            sl = pl.ds(fb * 16, 16)
            agg[ldx, sl] = jnp.maximum(agg[ldx, sl], rows[re, sl])
        return carry

      lax.fori_loop(0, GCH // 16, group_body, 0)

    def chunk_body(ci, pending):
      pltpu.sync_copy(src_hbm.at[pl.ds(ci * ECHUNK, ECHUNK)], srcv)
      pltpu.sync_copy(dst_hbm.at[pl.ds(ci * ECHUNK, ECHUNK)], dstv)

      def step(j, pending):
        off = j * 16
        d = dstv[pl.ds(off, 16)]
        sv = srcv[pl.ds(off, 16)]
        ld = d - lo
        m = (ld >= 0) & (ld < NPW)
        plsc.store_compressed(gidx.at[pl.ds(pending, 16)], sv, mask=m)
        plsc.store_compressed(gdst.at[pl.ds(pending, 16)], ld, mask=m)
        pending = pending + jnp.sum(m.astype(jnp.int32))
        do = pending >= GCH

        @pl.when(do)
        def _():
          flush(GCH)
          gidx[pl.ds(0, 16)] = gidx[pl.ds(GCH, 16)]
          gdst[pl.ds(0, 16)] = gdst[pl.ds(GCH, 16)]

        return jnp.where(do, pending - GCH, pending)

      return lax.fori_loop(0, NSTEPS, step, pending)

    pending = lax.fori_loop(0, E // ECHUNK, chunk_body, 0)
    flush(pending)
    pltpu.sync_copy(agg.at[pl.ds(0, NPW)], out_hbm.at[pl.ds(lo, NPW)])

  return segmax


_segmax128 = _make_segmax(128)
_segmax256 = _make_segmax(256)


def _mm_relu_kernel(x_ref, w_ref, b_ref, o_ref):
  acc = jnp.dot(x_ref[...], w_ref[...], preferred_element_type=jnp.float32)
  o_ref[...] = jnp.maximum(acc + b_ref[...], 0.0)


def _mm_relu(x, wt, b, block_rows):
  rows, fin = x.shape
  fout = wt.shape[1]
  grid = rows // block_rows
  return pl.pallas_call(
      _mm_relu_kernel,
      grid=(grid,),
      in_specs=[
          pl.BlockSpec((block_rows, fin), lambda i: (i, 0)),
          pl.BlockSpec((fin, fout), lambda i: (0, 0)),
          pl.BlockSpec((1, fout), lambda i: (0, 0)),
      ],
      out_specs=pl.BlockSpec((block_rows, fout), lambda i: (i, 0)),
      out_shape=jax.ShapeDtypeStruct((rows, fout), jnp.float32),
  )(x, wt, b)


def _fused_kernel(a_ref, wl_ref, bl_ref, wp_ref, bp_ref, o_ref):
  t = jnp.dot(a_ref[...], wl_ref[...], preferred_element_type=jnp.float32)
  t = jnp.maximum(t + bl_ref[...], 0.0)
  nrm = jnp.sqrt(jnp.sum(t * t, axis=1, keepdims=True))
  hn = t / jnp.maximum(nrm, 1e-12)
  acc = jnp.dot(hn, wp_ref[...], preferred_element_type=jnp.float32)
  o_ref[...] = jnp.maximum(acc + bp_ref[...], 0.0)


def _fused(a, wlt, bl, wpt, bp, block_rows=1024):
  rows, fin = a.shape
  fmid = wlt.shape[1]
  fout = wpt.shape[1]
  grid = rows // block_rows
  return pl.pallas_call(
      _fused_kernel,
      grid=(grid,),
      in_specs=[
          pl.BlockSpec((block_rows, fin), lambda i: (i, 0)),
          pl.BlockSpec((fin, fmid), lambda i: (0, 0)),
          pl.BlockSpec((1, fmid), lambda i: (0, 0)),
          pl.BlockSpec((fmid, fout), lambda i: (0, 0)),
          pl.BlockSpec((1, fout), lambda i: (0, 0)),
      ],
      out_specs=pl.BlockSpec((block_rows, fout), lambda i: (i, 0)),
      out_shape=jax.ShapeDtypeStruct((rows, fout), jnp.float32),
  )(a, wlt, bl, wpt, bp)


def _final_kernel(a_ref, w_ref, b_ref, o_ref):
  acc = jnp.dot(a_ref[...], w_ref[...], preferred_element_type=jnp.float32)
  o_ref[...] = acc + b_ref[...]


def _final(a, wt, b, block_rows=1024):
  rows, fin = a.shape
  fout = wt.shape[1]
  grid = rows // block_rows
  return pl.pallas_call(
      _final_kernel,
      grid=(grid,),
      in_specs=[
          pl.BlockSpec((block_rows, fin), lambda i: (i, 0)),
          pl.BlockSpec((fin, fout), lambda i: (0, 0)),
          pl.BlockSpec((1, fout), lambda i: (0, 0)),
      ],
      out_specs=pl.BlockSpec((block_rows, fout), lambda i: (i, 0)),
      out_shape=jax.ShapeDtypeStruct((rows, fout), jnp.float32),
  )(a, wt, b)


def kernel(node_feats, edge_index, W0p, b0p, W0, b0, bias0,
           W1p, b1p, W1, b1, bias1, W2p, b2p, W2, b2, bias2):
  src = edge_index[0].astype(jnp.int32)
  dst = edge_index[1].astype(jnp.int32)

  hp0 = _mm_relu(node_feats, W0p.T, b0p.reshape(1, -1), block_rows=1000)
  agg0 = _segmax128(hp0, src, dst)
  hp1 = _fused(agg0, W0.T, (b0 + bias0).reshape(1, -1),
               W1p.T, b1p.reshape(1, -1))
  agg1 = _segmax256(hp1, src, dst)
  hp2 = _fused(agg1, W1.T, (b1 + bias1).reshape(1, -1),
               W2p.T, b2p.reshape(1, -1))
  agg2 = _segmax256(hp2, src, dst)
  out = _final(agg2, W2.T, (b2 + bias2).reshape(1, -1))
  return out[:N]


# speedup vs baseline: 3.4585x; 2.5494x over previous
import functools

import jax
import jax.numpy as jnp
from jax import lax
from jax.experimental import pallas as pl
from jax.experimental.pallas import tpu as pltpu
from jax.experimental.pallas import tpu_sc as plsc

N = 10000
NPAD = 10240
NW = 32
NPW = NPAD // NW
E = 320000
ECHUNK = 16000
NSTEPS = ECHUNK // 16
FLUSH = 2048
LCAP = 327680
GCH = 128
ICAP = 16384

_params = pltpu.CompilerParams(
    needs_layout_passes=False, use_tc_tiling_on_sc=False)
_mesh = plsc.VectorSubcoreMesh(core_axis_name="c", subcore_axis_name="s")


def _lane(vec, i, iot):
  return jnp.max(jnp.where(iot == i, vec, jnp.int32(-1)))


@functools.partial(
    pl.kernel,
    out_type=(
        jax.ShapeDtypeStruct((NW, LCAP), jnp.int32),
        jax.ShapeDtypeStruct((NW, LCAP), jnp.int32),
        jax.ShapeDtypeStruct((NW, 16), jnp.int32),
    ),
    mesh=_mesh,
    scratch_types=[
        pltpu.VMEM((ECHUNK,), jnp.int32),
        pltpu.VMEM((ECHUNK,), jnp.int32),
        pltpu.VMEM((FLUSH + 16,), jnp.int32),
        pltpu.VMEM((FLUSH + 16,), jnp.int32),
        pltpu.VMEM((16,), jnp.int32),
    ],
    compiler_params=_params,
)
def _partition(src_hbm, dst_hbm, esrc, eldst, counts,
               srcv, dstv, sbuf, dbuf, cbuf):
  c = lax.axis_index("c")
  s = lax.axis_index("s")
  wid = s * 2 + c
  lo = wid * NPW

  zi = jnp.zeros((16,), jnp.int32)
  for i in range((FLUSH + 16) // 16):
    sbuf[pl.ds(i * 16, 16)] = zi

  def chunk_body(ci, carry):
    pltpu.sync_copy(src_hbm.at[pl.ds(ci * ECHUNK, ECHUNK)], srcv)
    pltpu.sync_copy(dst_hbm.at[pl.ds(ci * ECHUNK, ECHUNK)], dstv)

    def step(j, carry):
      bufn, out_off = carry
      off = j * 16
      d = dstv[pl.ds(off, 16)]
      sv = srcv[pl.ds(off, 16)]
      ld = d - lo
      m = (ld >= 0) & (ld < NPW)
      plsc.store_compressed(sbuf.at[pl.ds(bufn, 16)], sv, mask=m)
      plsc.store_compressed(dbuf.at[pl.ds(bufn, 16)], ld, mask=m)
      bufn = bufn + jnp.sum(m.astype(jnp.int32))
      do = bufn >= FLUSH

      @pl.when(do)
      def _():
        oo = pl.multiple_of(out_off, FLUSH)
        pltpu.sync_copy(sbuf.at[pl.ds(0, FLUSH)],
                        esrc.at[wid, pl.ds(oo, FLUSH)])
        pltpu.sync_copy(dbuf.at[pl.ds(0, FLUSH)],
                        eldst.at[wid, pl.ds(oo, FLUSH)])
        sbuf[pl.ds(0, 16)] = sbuf[pl.ds(FLUSH, 16)]
        dbuf[pl.ds(0, 16)] = dbuf[pl.ds(FLUSH, 16)]

      bufn = jnp.where(do, bufn - FLUSH, bufn)
      out_off = jnp.where(do, out_off + FLUSH, out_off)
      return bufn, out_off

    return lax.fori_loop(0, NSTEPS, step, carry)

  bufn, out_off = lax.fori_loop(0, E // ECHUNK, chunk_body,
                                (jnp.int32(0), jnp.int32(0)))
  oo = pl.multiple_of(out_off, FLUSH)
  pltpu.sync_copy(sbuf.at[pl.ds(0, FLUSH)],
                  esrc.at[wid, pl.ds(oo, FLUSH)])
  pltpu.sync_copy(dbuf.at[pl.ds(0, FLUSH)],
                  eldst.at[wid, pl.ds(oo, FLUSH)])
  cbuf[pl.ds(0, 16)] = jnp.broadcast_to(out_off + bufn, (16,)).astype(jnp.int32)
  pltpu.sync_copy(cbuf, counts.at[wid])


def _make_gather_max(F):
  @functools.partial(
      pl.kernel,
      out_type=jax.ShapeDtypeStruct((NPAD, F), jnp.bfloat16),
      mesh=_mesh,
      scratch_types=[
          pltpu.VMEM((NPW + 1, F), jnp.bfloat16),
          pltpu.VMEM((ICAP,), jnp.int32),
          pltpu.VMEM((ICAP,), jnp.int32),
          pltpu.VMEM((2, GCH, F), jnp.bfloat16),
          pltpu.VMEM((16,), jnp.int32),
          pltpu.SemaphoreType.DMA,
          pltpu.SemaphoreType.DMA,
      ],
      compiler_params=_params,
  )
  def gather_max(hp_hbm, esrc, eldst, counts, out_hbm,
                 agg, sstage, dstage, rows, cntv, sem0, sem1):
    c = lax.axis_index("c")
    s = lax.axis_index("s")
    wid = s * 2 + c
    lo = wid * NPW
    iot = lax.iota(jnp.int32, 16)

    zf = jnp.zeros((32,), jnp.bfloat16)

    def zero_row(r, carry):
      for fb in range(F // 32):
        agg[r, pl.ds(fb * 32, 32)] = zf
      return carry

    lax.fori_loop(0, NPW + 1, zero_row, 0)

    pltpu.sync_copy(counts.at[wid], cntv)
    cnt = _lane(cntv[pl.ds(0, 16)], 0, iot)
    nbatches = (cnt + GCH - 1) // GCH
    nchunks = (cnt + ICAP - 1) // ICAP
    bpc = ICAP // GCH

    def start_gather(slot_rows, boff, sem):
      boff = pl.multiple_of(boff, GCH)
      pltpu.make_async_copy(
          hp_hbm.at[sstage.at[pl.ds(boff, GCH)]], slot_rows, sem).start()

    def wait_gather(slot_rows, boff, sem):
      boff = pl.multiple_of(boff, GCH)
      pltpu.make_async_copy(
          hp_hbm.at[sstage.at[pl.ds(boff, GCH)]], slot_rows, sem).wait()

    def process(slot_rows, b, nvalid):
      def group_body(g, carry):
        dv = dstage[pl.ds(b * GCH + g * 16, 16)]
        base = g * 16
        for e in range(16):
          ldx = _lane(dv, e, iot)
          ldx = jnp.where(base + e < nvalid, ldx, jnp.int32(NPW))
          re = base + e
          for fb in range(F // 32):
            sl = pl.ds(fb * 32, 32)
            agg[ldx, sl] = jnp.maximum(agg[ldx, sl], slot_rows[re, sl])
        return carry

      lax.fori_loop(0, GCH // 16, group_body, 0)

    def chunk_body(ci, carry):
      co = pl.multiple_of(ci * ICAP, ICAP)
      pltpu.sync_copy(esrc.at[wid, pl.ds(co, ICAP)], sstage)
      pltpu.sync_copy(eldst.at[wid, pl.ds(co, ICAP)], dstage)
      nb_here = jnp.minimum(nbatches - ci * bpc, bpc)

      @pl.when(nb_here > 0)
      def _():
        start_gather(rows.at[0], 0, sem0)

      def pair_body(i2, carry):
        for p, sem_p, sem_q in ((0, sem0, sem1), (1, sem1, sem0)):
          b = i2 * 2 + p
          gb = ci * bpc + b

          @pl.when(b + 1 < nb_here)
          def _():
            start_gather(rows.at[1 - p], (b + 1) * GCH, sem_q)

          @pl.when(b < nb_here)
          def _():
            wait_gather(rows.at[p], b * GCH, sem_p)
            nvalid = jnp.minimum(cnt - gb * GCH, GCH)
            process(rows.at[p], b, nvalid)
        return carry

      npairs = (nb_here + 1) // 2
      lax.fori_loop(0, npairs, pair_body, 0)
      return carry

    lax.fori_loop(0, nchunks, chunk_body, 0)
    pltpu.sync_copy(agg.at[pl.ds(0, NPW)], out_hbm.at[pl.ds(lo, NPW)])

  return gather_max


_gather128 = _make_gather_max(128)
_gather256 = _make_gather_max(256)


def _mm_relu_kernel(x_ref, w_ref, b_ref, o_ref):
  acc = jnp.dot(x_ref[...], w_ref[...], preferred_element_type=jnp.float32)
  o_ref[...] = jnp.maximum(acc + b_ref[...], 0.0).astype(jnp.bfloat16)


def _mm_relu(x, wt, b, block_rows):
  rows, fin = x.shape
  fout = wt.shape[1]
  grid = rows // block_rows
  return pl.pallas_call(
      _mm_relu_kernel,
      grid=(grid,),
      in_specs=[
          pl.BlockSpec((block_rows, fin), lambda i: (i, 0)),
          pl.BlockSpec((fin, fout), lambda i: (0, 0)),
          pl.BlockSpec((1, fout), lambda i: (0, 0)),
      ],
      out_specs=pl.BlockSpec((block_rows, fout), lambda i: (i, 0)),
      out_shape=jax.ShapeDtypeStruct((rows, fout), jnp.bfloat16),
  )(x, wt, b)


def _fused_kernel(a_ref, wl_ref, bl_ref, wp_ref, bp_ref, o_ref):
  a32 = a_ref[...].astype(jnp.float32)
  t = jnp.dot(a32, wl_ref[...], preferred_element_type=jnp.float32)
  t = jnp.maximum(t + bl_ref[...], 0.0)
  nrm = jnp.sqrt(jnp.sum(t * t, axis=1, keepdims=True))
  hn = t / jnp.maximum(nrm, 1e-12)
  acc = jnp.dot(hn, wp_ref[...], preferred_element_type=jnp.float32)
  o_ref[...] = jnp.maximum(acc + bp_ref[...], 0.0).astype(jnp.bfloat16)


def _fused(a, wlt, bl, wpt, bp, block_rows=1024):
  rows, fin = a.shape
  fmid = wlt.shape[1]
  fout = wpt.shape[1]
  grid = rows // block_rows
  return pl.pallas_call(
      _fused_kernel,
      grid=(grid,),
      in_specs=[
          pl.BlockSpec((block_rows, fin), lambda i: (i, 0)),
          pl.BlockSpec((fin, fmid), lambda i: (0, 0)),
          pl.BlockSpec((1, fmid), lambda i: (0, 0)),
          pl.BlockSpec((fmid, fout), lambda i: (0, 0)),
          pl.BlockSpec((1, fout), lambda i: (0, 0)),
      ],
      out_specs=pl.BlockSpec((block_rows, fout), lambda i: (i, 0)),
      out_shape=jax.ShapeDtypeStruct((rows, fout), jnp.bfloat16),
  )(a, wlt, bl, wpt, bp)


def _final_kernel(a_ref, w_ref, b_ref, o_ref):
  a32 = a_ref[...].astype(jnp.float32)
  acc = jnp.dot(a32, w_ref[...], preferred_element_type=jnp.float32)
  o_ref[...] = acc + b_ref[...]


def _final(a, wt, b, block_rows=1024):
  rows, fin = a.shape
  fout = wt.shape[1]
  grid = rows // block_rows
  return pl.pallas_call(
      _final_kernel,
      grid=(grid,),
      in_specs=[
          pl.BlockSpec((block_rows, fin), lambda i: (i, 0)),
          pl.BlockSpec((fin, fout), lambda i: (0, 0)),
          pl.BlockSpec((1, fout), lambda i: (0, 0)),
      ],
      out_specs=pl.BlockSpec((block_rows, fout), lambda i: (i, 0)),
      out_shape=jax.ShapeDtypeStruct((rows, fout), jnp.float32),
  )(a, wt, b)


def kernel(node_feats, edge_index, W0p, b0p, W0, b0, bias0,
           W1p, b1p, W1, b1, bias1, W2p, b2p, W2, b2, bias2):
  src = edge_index[0].astype(jnp.int32)
  dst = edge_index[1].astype(jnp.int32)

  esrc, eldst, counts = _partition(src, dst)
  hp0 = _mm_relu(node_feats, W0p.T, b0p.reshape(1, -1), block_rows=1000)
  agg0 = _gather128(hp0, esrc, eldst, counts)
  hp1 = _fused(agg0, W0.T, (b0 + bias0).reshape(1, -1),
               W1p.T, b1p.reshape(1, -1))
  agg1 = _gather256(hp1, esrc, eldst, counts)
  hp2 = _fused(agg1, W1.T, (b1 + bias1).reshape(1, -1),
               W2p.T, b2p.reshape(1, -1))
  agg2 = _gather256(hp2, esrc, eldst, counts)
  out = _final(agg2, W2.T, (b2 + bias2).reshape(1, -1))
  return out[:N]


# speedup vs baseline: 3.9657x; 1.1466x over previous
import functools

import jax
import jax.numpy as jnp
from jax import lax
from jax.experimental import pallas as pl
from jax.experimental.pallas import tpu as pltpu
from jax.experimental.pallas import tpu_sc as plsc

N = 10000
NPAD = 10240
NW = 32
NPW = NPAD // NW
E = 320000
ECHUNK = 16000
NSTEPS = ECHUNK // 16
FLUSH = 2048
LCAP = 327680
GCH = 128
ICAP = 16384

_params = pltpu.CompilerParams(
    needs_layout_passes=False, use_tc_tiling_on_sc=False)
_mesh = plsc.VectorSubcoreMesh(core_axis_name="c", subcore_axis_name="s")


def _lane(vec, i, iot):
  return jnp.max(jnp.where(iot == i, vec, jnp.int32(-1)))


@functools.partial(
    pl.kernel,
    out_type=(
        jax.ShapeDtypeStruct((NW, LCAP), jnp.int32),
        jax.ShapeDtypeStruct((NW, 16), jnp.int32),
    ),
    mesh=_mesh,
    scratch_types=[
        pltpu.VMEM((ECHUNK,), jnp.int32),
        pltpu.VMEM((ECHUNK,), jnp.int32),
        pltpu.VMEM((FLUSH + 144,), jnp.int32),
        pltpu.VMEM((16,), jnp.int32),
    ],
    compiler_params=_params,
)
def _partition(src_hbm, dst_hbm, epack, counts,
               srcv, dstv, sbuf, cbuf):
  c = lax.axis_index("c")
  s = lax.axis_index("s")
  wid = s * 2 + c
  lo = wid * NPW

  zi = jnp.zeros((16,), jnp.int32)
  for i in range((FLUSH + 144) // 16):
    sbuf[pl.ds(i * 16, 16)] = zi

  def chunk_body(ci, carry):
    pltpu.sync_copy(src_hbm.at[pl.ds(ci * ECHUNK, ECHUNK)], srcv)
    pltpu.sync_copy(dst_hbm.at[pl.ds(ci * ECHUNK, ECHUNK)], dstv)

    def superstep(j, carry):
      bufn_vec, out_off = carry
      for k in range(8):
        off = (j * 8 + k) * 16
        d = dstv[pl.ds(off, 16)]
        sv = srcv[pl.ds(off, 16)]
        ld = d - lo
        m = (ld >= 0) & (ld < NPW)
        packed = sv | (ld << 14)
        cs = plsc.cumsum(m.astype(jnp.int32))
        pos = bufn_vec + cs - 1
        plsc.store_scatter(sbuf, [pos], packed, mask=m)
        bufn_vec = bufn_vec + plsc.all_reduce_population_count(m)
      bufn_s = jnp.max(bufn_vec)
      do = bufn_s >= FLUSH

      @pl.when(do)
      def _():
        oo = pl.multiple_of(out_off, FLUSH)
        pltpu.sync_copy(sbuf.at[pl.ds(0, FLUSH)],
                        epack.at[wid, pl.ds(oo, FLUSH)])
        for t in range(8):
          sbuf[pl.ds(t * 16, 16)] = sbuf[pl.ds(FLUSH + t * 16, 16)]

      bufn_vec = jnp.where(do, bufn_vec - FLUSH, bufn_vec)
      out_off = jnp.where(do, out_off + FLUSH, out_off)
      return bufn_vec, out_off

    return lax.fori_loop(0, NSTEPS // 8, superstep, carry)

  bufn_vec, out_off = lax.fori_loop(
      0, E // ECHUNK, chunk_body,
      (jnp.zeros((16,), jnp.int32), jnp.int32(0)))
  bufn = jnp.max(bufn_vec)
  oo = pl.multiple_of(out_off, FLUSH)
  pltpu.sync_copy(sbuf.at[pl.ds(0, FLUSH)],
                  epack.at[wid, pl.ds(oo, FLUSH)])
  cbuf[pl.ds(0, 16)] = jnp.broadcast_to(out_off + bufn, (16,)).astype(jnp.int32)
  pltpu.sync_copy(cbuf, counts.at[wid])


def _make_gather_max(F):
  @functools.partial(
      pl.kernel,
      out_type=jax.ShapeDtypeStruct((NPAD, F), jnp.bfloat16),
      mesh=_mesh,
      scratch_types=[
          pltpu.VMEM((NPW + 1, F), jnp.bfloat16),
          pltpu.VMEM((ICAP,), jnp.int32),
          pltpu.VMEM((2, GCH), jnp.int32),
          pltpu.VMEM((2, GCH), jnp.int32),
          pltpu.VMEM((2, GCH, F), jnp.bfloat16),
          pltpu.VMEM((16,), jnp.int32),
          pltpu.SemaphoreType.DMA,
          pltpu.SemaphoreType.DMA,
      ],
      compiler_params=_params,
  )
  def gather_max(hp_hbm, epack, counts, out_hbm,
                 agg, pstage, sslot, dslot, rows, cntv, sem0, sem1):
    c = lax.axis_index("c")
    s = lax.axis_index("s")
    wid = s * 2 + c
    lo = wid * NPW
    iot = lax.iota(jnp.int32, 16)

    zf = jnp.zeros((32,), jnp.bfloat16)

    def zero_row(r, carry):
      for fb in range(F // 32):
        agg[r, pl.ds(fb * 32, 32)] = zf
      return carry

    lax.fori_loop(0, NPW + 1, zero_row, 0)

    pltpu.sync_copy(counts.at[wid], cntv)
    cnt = _lane(cntv[pl.ds(0, 16)], 0, iot)
    nbatches = (cnt + GCH - 1) // GCH
    nchunks = (cnt + ICAP - 1) // ICAP
    bpc = ICAP // GCH

    def unpack_batch(slot, boff):
      boff = pl.multiple_of(boff, GCH)
      for u in range(GCH // 16):
        v = pstage[pl.ds(boff + u * 16, 16)]
        sslot[slot, pl.ds(u * 16, 16)] = v & jnp.int32(0x3FFF)
        dslot[slot, pl.ds(u * 16, 16)] = lax.shift_right_logical(
            v, jnp.int32(14))

    def start_gather(slot, slot_rows, sem):
      pltpu.make_async_copy(
          hp_hbm.at[sslot.at[slot]], slot_rows, sem).start()

    def wait_gather(slot, slot_rows, sem):
      pltpu.make_async_copy(
          hp_hbm.at[sslot.at[slot]], slot_rows, sem).wait()

    def process(slot, slot_rows, nvalid):
      def group_body(g, carry):
        dv = dslot[slot, pl.ds(g * 16, 16)]
        base = g * 16
        ldxs = []
        for e in range(16):
          ldx = _lane(dv, e, iot)
          ldxs.append(jnp.where(base + e < nvalid, ldx, jnp.int32(NPW)))
        for e in range(16):
          re = base + e
          for fb in range(F // 32):
            sl = pl.ds(fb * 32, 32)
            agg[ldxs[e], sl] = jnp.maximum(agg[ldxs[e], sl],
                                           slot_rows[re, sl])
        return carry

      lax.fori_loop(0, GCH // 16, group_body, 0)

    def chunk_body(ci, carry):
      co = pl.multiple_of(ci * ICAP, ICAP)
      pltpu.sync_copy(epack.at[wid, pl.ds(co, ICAP)], pstage)
      nb_here = jnp.minimum(nbatches - ci * bpc, bpc)

      @pl.when(nb_here > 0)
      def _():
        unpack_batch(0, 0)
        start_gather(0, rows.at[0], sem0)

      def pair_body(i2, carry):
        for p, sem_p, sem_q in ((0, sem0, sem1), (1, sem1, sem0)):
          b = i2 * 2 + p
          gb = ci * bpc + b

          @pl.when(b + 1 < nb_here)
          def _():
            unpack_batch(1 - p, (b + 1) * GCH)
            start_gather(1 - p, rows.at[1 - p], sem_q)

          @pl.when(b < nb_here)
          def _():
            wait_gather(p, rows.at[p], sem_p)
            nvalid = jnp.minimum(cnt - gb * GCH, GCH)
            process(p, rows.at[p], nvalid)
        return carry

      npairs = (nb_here + 1) // 2
      lax.fori_loop(0, npairs, pair_body, 0)
      return carry

    lax.fori_loop(0, nchunks, chunk_body, 0)
    pltpu.sync_copy(agg.at[pl.ds(0, NPW)], out_hbm.at[pl.ds(lo, NPW)])

  return gather_max


_gather128 = _make_gather_max(128)
_gather256 = _make_gather_max(256)


def _mm_relu_kernel(x_ref, w_ref, b_ref, o_ref):
  acc = jnp.dot(x_ref[...], w_ref[...], preferred_element_type=jnp.float32)
  o_ref[...] = jnp.maximum(acc + b_ref[...], 0.0).astype(jnp.bfloat16)


def _mm_relu(x, wt, b, block_rows):
  rows, fin = x.shape
  fout = wt.shape[1]
  grid = rows // block_rows
  return pl.pallas_call(
      _mm_relu_kernel,
      grid=(grid,),
      in_specs=[
          pl.BlockSpec((block_rows, fin), lambda i: (i, 0)),
          pl.BlockSpec((fin, fout), lambda i: (0, 0)),
          pl.BlockSpec((1, fout), lambda i: (0, 0)),
      ],
      out_specs=pl.BlockSpec((block_rows, fout), lambda i: (i, 0)),
      out_shape=jax.ShapeDtypeStruct((rows, fout), jnp.bfloat16),
  )(x, wt, b)


def _fused_kernel(a_ref, wl_ref, bl_ref, wp_ref, bp_ref, o_ref):
  a32 = a_ref[...].astype(jnp.float32)
  t = jnp.dot(a32, wl_ref[...], preferred_element_type=jnp.float32)
  t = jnp.maximum(t + bl_ref[...], 0.0)
  nrm = jnp.sqrt(jnp.sum(t * t, axis=1, keepdims=True))
  hn = t / jnp.maximum(nrm, 1e-12)
  acc = jnp.dot(hn, wp_ref[...], preferred_element_type=jnp.float32)
  o_ref[...] = jnp.maximum(acc + bp_ref[...], 0.0).astype(jnp.bfloat16)


def _fused(a, wlt, bl, wpt, bp, block_rows=1024):
  rows, fin = a.shape
  fmid = wlt.shape[1]
  fout = wpt.shape[1]
  grid = rows // block_rows
  return pl.pallas_call(
      _fused_kernel,
      grid=(grid,),
      in_specs=[
          pl.BlockSpec((block_rows, fin), lambda i: (i, 0)),
          pl.BlockSpec((fin, fmid), lambda i: (0, 0)),
          pl.BlockSpec((1, fmid), lambda i: (0, 0)),
          pl.BlockSpec((fmid, fout), lambda i: (0, 0)),
          pl.BlockSpec((1, fout), lambda i: (0, 0)),
      ],
      out_specs=pl.BlockSpec((block_rows, fout), lambda i: (i, 0)),
      out_shape=jax.ShapeDtypeStruct((rows, fout), jnp.bfloat16),
  )(a, wlt, bl, wpt, bp)


def _final_kernel(a_ref, w_ref, b_ref, o_ref):
  a32 = a_ref[...].astype(jnp.float32)
  acc = jnp.dot(a32, w_ref[...], preferred_element_type=jnp.float32)
  o_ref[...] = acc + b_ref[...]


def _final(a, wt, b, block_rows=1024):
  rows, fin = a.shape
  fout = wt.shape[1]
  grid = rows // block_rows
  return pl.pallas_call(
      _final_kernel,
      grid=(grid,),
      in_specs=[
          pl.BlockSpec((block_rows, fin), lambda i: (i, 0)),
          pl.BlockSpec((fin, fout), lambda i: (0, 0)),
          pl.BlockSpec((1, fout), lambda i: (0, 0)),
      ],
      out_specs=pl.BlockSpec((block_rows, fout), lambda i: (i, 0)),
      out_shape=jax.ShapeDtypeStruct((rows, fout), jnp.float32),
  )(a, wt, b)


def kernel(node_feats, edge_index, W0p, b0p, W0, b0, bias0,
           W1p, b1p, W1, b1, bias1, W2p, b2p, W2, b2, bias2):
  src = edge_index[0].astype(jnp.int32)
  dst = edge_index[1].astype(jnp.int32)

  epack, counts = _partition(src, dst)
  hp0 = _mm_relu(node_feats, W0p.T, b0p.reshape(1, -1), block_rows=1000)
  agg0 = _gather128(hp0, epack, counts)
  hp1 = _fused(agg0, W0.T, (b0 + bias0).reshape(1, -1),
               W1p.T, b1p.reshape(1, -1))
  agg1 = _gather256(hp1, epack, counts)
  hp2 = _fused(agg1, W1.T, (b1 + bias1).reshape(1, -1),
               W2p.T, b2p.reshape(1, -1))
  agg2 = _gather256(hp2, epack, counts)
  out = _final(agg2, W2.T, (b2 + bias2).reshape(1, -1))
  return out[:N]
